# baseline (speedup 1.0000x reference)
import jax
import jax.numpy as jnp
import numpy as np
from jax.experimental import pallas as pl
from jax.experimental.pallas import tpu as pltpu

_N = 256
_C = 3
_S = 8
_T = 3


def _pool_mats() -> np.ndarray:
    mats = []
    for lvl in range(_T + 1):
        k = 1 << lvl
        idx = np.arange(_N) // k
        mats.append((idx[:, None] == idx[None, :]).astype(np.float32) / k)
    return np.stack(mats)


_A_STACK = _pool_mats()


def _body(t_ref, a_ref, x_ref, o_ref):
    b = pl.program_id(0)
    for i in range(_S):
        tv = t_ref[b * _S + i]
        a = a_ref[tv]
        for ch in range(_C):
            y = jnp.dot(x_ref[i, ch], a, preferred_element_type=jnp.float32)
            o_ref[i, ch] = jnp.dot(a, y, preferred_element_type=jnp.float32)


def kernel(x0, t):
    batch = x0.shape[0]
    t32 = t.astype(jnp.int32)
    a_stack = jnp.asarray(_A_STACK)
    grid_spec = pltpu.PrefetchScalarGridSpec(
        num_scalar_prefetch=1,
        grid=(batch // _S,),
        in_specs=[
            pl.BlockSpec((_T + 1, _N, _N), lambda b, tref: (0, 0, 0)),
            pl.BlockSpec((_S, _C, _N, _N), lambda b, tref: (b, 0, 0, 0)),
        ],
        out_specs=pl.BlockSpec((_S, _C, _N, _N), lambda b, tref: (b, 0, 0, 0)),
    )
    return pl.pallas_call(
        _body,
        out_shape=jax.ShapeDtypeStruct(x0.shape, x0.dtype),
        grid_spec=grid_spec,
        compiler_params=pltpu.CompilerParams(
            dimension_semantics=("parallel",),
            vmem_limit_bytes=56 * 1024 * 1024,
        ),
        name="superres_pool_upsample",
    )(t32, a_stack, x0)


# speedup vs baseline: 1.0103x; 1.0103x over previous
import jax
import jax.numpy as jnp
import numpy as np
from jax.experimental import pallas as pl
from jax.experimental.pallas import tpu as pltpu

_N = 256
_C = 3
_S = 16
_T = 3


def _pool_mats() -> np.ndarray:
    mats = []
    for lvl in range(_T + 1):
        k = 1 << lvl
        idx = np.arange(_N) // k
        mats.append((idx[:, None] == idx[None, :]).astype(np.float32) / k)
    return np.stack(mats)


_A_STACK = _pool_mats()


def _body(t_ref, a_ref, x_ref, o_ref):
    b = pl.program_id(0)
    for i in range(_S):
        tv = t_ref[b * _S + i]
        a = a_ref[tv]
        for ch in range(_C):
            y = jnp.dot(x_ref[i, ch], a, preferred_element_type=jnp.float32)
            o_ref[i, ch] = jnp.dot(a, y, preferred_element_type=jnp.float32)


def kernel(x0, t):
    batch = x0.shape[0]
    t32 = t.astype(jnp.int32)
    a_stack = jnp.asarray(_A_STACK)
    grid_spec = pltpu.PrefetchScalarGridSpec(
        num_scalar_prefetch=1,
        grid=(batch // _S,),
        in_specs=[
            pl.BlockSpec((_T + 1, _N, _N), lambda b, tref: (0, 0, 0)),
            pl.BlockSpec((_S, _C, _N, _N), lambda b, tref: (b, 0, 0, 0)),
        ],
        out_specs=pl.BlockSpec((_S, _C, _N, _N), lambda b, tref: (b, 0, 0, 0)),
    )
    return pl.pallas_call(
        _body,
        out_shape=jax.ShapeDtypeStruct(x0.shape, x0.dtype),
        grid_spec=grid_spec,
        compiler_params=pltpu.CompilerParams(
            dimension_semantics=("parallel",),
            vmem_limit_bytes=56 * 1024 * 1024,
        ),
        name="superres_pool_upsample",
    )(t32, a_stack, x0)
